# baseline (speedup 1.0000x reference)
import functools

import jax
import jax.numpy as jnp
from jax import lax
from jax.experimental import pallas as pl
from jax.experimental.pallas import tpu as pltpu
from jax.experimental.pallas import tpu_sc as plsc

N_CLS = 1000
N_CTX = 16
CTX_DIM = 512
CONTEXT_LEN = 77
N_SUFFIX = CONTEXT_LEN - 1 - N_CTX

NUM_WORKERS = 32
CB = 32
LAST_CBASE = N_CLS - CB
NBUF = 2
CHUNK_SLOTS = 3

_A_TOKENS = [0] + list(range(1 + N_CTX, CONTEXT_LEN))
_B_TOKENS = list(range(1, 1 + N_CTX))
_SLOT_TOKENS = _A_TOKENS + _B_TOKENS
_CHUNKS = []
for _s in range(0, 61, CHUNK_SLOTS):
    _CHUNKS.append(("table", _s, min(CHUNK_SLOTS, 61 - _s)))
for _s in range(61, 77, CHUNK_SLOTS):
    _CHUNKS.append(("ctx", _s, min(CHUNK_SLOTS, 77 - _s)))
NCHUNK = len(_CHUNKS)


def _sc_prompt_assemble(ctx2d, table, tokens):
    mesh = plsc.VectorSubcoreMesh(core_axis_name="c", subcore_axis_name="s")

    @functools.partial(
        pl.kernel,
        mesh=mesh,
        out_type=jax.ShapeDtypeStruct((CONTEXT_LEN * N_CLS, CTX_DIM), jnp.float32),
        scratch_types=[
            pltpu.VMEM((CONTEXT_LEN * CB,), jnp.int32),
            pltpu.VMEM((CHUNK_SLOTS * CB, CTX_DIM), jnp.float32),
            pltpu.VMEM((CHUNK_SLOTS * CB, CTX_DIM), jnp.float32),
            pltpu.SemaphoreType.DMA,
            pltpu.SemaphoreType.DMA,
            pltpu.SemaphoreType.DMA,
            pltpu.SemaphoreType.DMA,
        ],
    )
    def k(ctx_hbm, table_hbm, tokt_hbm, out_hbm,
          idx_v, buf0, buf1,
          gsem0, gsem1, ssem0, ssem1):
        bufs = (buf0, buf1)
        gsems = (gsem0, gsem1)
        ssems = (ssem0, ssem1)
        wid = lax.axis_index("s") * 2 + lax.axis_index("c")
        cbase = jnp.minimum(wid * CB, LAST_CBASE)

        idx_cps = [
            pltpu.async_copy(
                tokt_hbm.at[pl.ds(t * N_CLS + cbase, CB)],
                idx_v.at[pl.ds(s * CB, CB)],
                gsem0,
            )
            for s, t in enumerate(_A_TOKENS)
        ]
        lane = lax.iota(jnp.int32, 16)
        for s, t in enumerate(_SLOT_TOKENS):
            if s < len(_A_TOKENS):
                continue
            for h in range(2):
                vals = (cbase + lane + 16 * h) * N_CTX + (t - 1)
                idx_v[pl.ds(s * CB + 16 * h, 16)] = vals
        for cp in idx_cps:
            cp.wait()

        def chunk_rows(ci):
            return _CHUNKS[ci][2] * CB

        def src_ref(ci):
            return ctx_hbm if _CHUNKS[ci][0] == "ctx" else table_hbm

        def start(ci, b):
            n = chunk_rows(ci)
            s0 = _CHUNKS[ci][1]
            pltpu.async_copy(
                src_ref(ci).at[idx_v.at[pl.ds(s0 * CB, n)]],
                bufs[b].at[pl.ds(0, n)],
                gsems[b],
            )

        def finish(ci, b):
            n = chunk_rows(ci)
            s0 = _CHUNKS[ci][1]
            pltpu.make_async_copy(
                src_ref(ci).at[idx_v.at[pl.ds(s0 * CB, n)]],
                bufs[b].at[pl.ds(0, n)],
                gsems[b],
            ).wait()
            for kk in range(n // CB):
                t = _SLOT_TOKENS[s0 + kk]
                pltpu.async_copy(
                    bufs[b].at[pl.ds(kk * CB, CB)],
                    out_hbm.at[pl.ds(t * N_CLS + cbase, CB)],
                    ssems[b],
                )

        def drain(ci, b):
            n = chunk_rows(ci)
            s0 = _CHUNKS[ci][1]
            for kk in range(n // CB):
                t = _SLOT_TOKENS[s0 + kk]
                pltpu.make_async_copy(
                    bufs[b].at[pl.ds(kk * CB, CB)],
                    out_hbm.at[pl.ds(t * N_CLS + cbase, CB)],
                    ssems[b],
                ).wait()

        for s in range(NCHUNK + 2):
            if s - 2 >= 0:
                drain(s - 2, (s - 2) % NBUF)
            if s < NCHUNK:
                start(s, s % NBUF)
            if 0 <= s - 1 < NCHUNK:
                finish(s - 1, (s - 1) % NBUF)

    return k(ctx2d, table, tokens)


def kernel(ctx, table, tokens):
    ctx2d = ctx.reshape(N_CLS * N_CTX, CTX_DIM)
    tokens_t = tokens.T.reshape(-1)
    out2d = _sc_prompt_assemble(ctx2d, table, tokens_t)
    return out2d.reshape(CONTEXT_LEN, N_CLS, CTX_DIM).transpose(1, 0, 2)
